# baseline (speedup 1.0000x reference)
import jax
import jax.numpy as jnp
from jax import lax
from jax.experimental import pallas as pl
from jax.experimental.pallas import tpu as pltpu
from jax.experimental.pallas import tpu_sc as plsc

VOCAB = 1000000
ROWS = 4096
CHUNK_SIZE = 16
N_CHUNKS = 4
BATCH = 16384
VPAD = 1000064

NC = 2
NS = 16
NW = NC * NS
BPW = BATCH // NW
PW = BPW * N_CHUNKS
NSLICE = PW // 128


def _body(x_hbm, h01_hbm, t0_hbm, t1_hbm, out_hbm,
          xv, hidx, cw, ti0, ti1, g, sem):
    wid = lax.axis_index("s") * NC + lax.axis_index("c")
    base = wid * BPW

    pltpu.sync_copy(x_hbm.at[pl.ds(base, BPW)], xv)

    def hidx_body(k, _):
        xq = xv[pl.ds(k * 16, 16)]
        hv = (lax.shift_left(lax.shift_right_logical(xq, 7), 9)
              + lax.bitwise_and(xq, 127))
        for c in range(N_CHUNKS):
            hidx[pl.ds(c * BPW + k * 16, 16)] = hv + c * 128
        return 0
    lax.fori_loop(0, BPW // 16, hidx_body, 0, unroll=2)

    copies = []
    for j in range(NSLICE):
        idx = hidx.at[pl.ds(j * 128, 128)]
        copies.append(pltpu.async_copy(
            h01_hbm.at[idx], cw.at[pl.ds(j * 128, 128)], sem))
    for cp in copies:
        cp.wait()

    def unpack_body(k, _):
        sl = pl.ds(k * 16, 16)
        w = cw[sl]
        ti0[sl] = lax.bitwise_and(w, 0xFFFF)
        ti1[sl] = lax.shift_right_logical(w, 16)
        return 0
    lax.fori_loop(0, PW // 16, unpack_body, 0, unroll=2)

    copies = []
    for j in range(NSLICE):
        copies.append(pltpu.async_copy(
            t0_hbm.at[ti0.at[pl.ds(j * 128, 128)]],
            g.at[pl.ds(j * 128, 128)], sem))
    for cp in copies:
        cp.wait()

    copies = []
    for j in range(NSLICE):
        copies.append(pltpu.async_copy(
            t1_hbm.at[ti1.at[pl.ds(j * 128, 128)]],
            g.at[pl.ds(j * 128, 128)], sem, add=True))
    for cp in copies:
        cp.wait()

    for c in range(N_CHUNKS):
        pltpu.sync_copy(g.at[pl.ds(c * BPW, BPW), :],
                        out_hbm.at[pl.ds(base, BPW),
                                   pl.ds(c * CHUNK_SIZE, CHUNK_SIZE)])


@jax.jit
def _run(x, h01p, t0f, t1f):
    mesh = plsc.VectorSubcoreMesh(core_axis_name="c", subcore_axis_name="s")
    f = pl.kernel(
        _body,
        out_type=jax.ShapeDtypeStruct((BATCH, N_CHUNKS * CHUNK_SIZE),
                                      jnp.float32),
        mesh=mesh,
        scratch_types=[
            pltpu.VMEM((BPW,), jnp.int32),
            pltpu.VMEM((PW,), jnp.int32),
            pltpu.VMEM((PW,), jnp.int32),
            pltpu.VMEM((PW,), jnp.int32),
            pltpu.VMEM((PW,), jnp.int32),
            pltpu.VMEM((PW, CHUNK_SIZE), jnp.float32),
            pltpu.SemaphoreType.DMA,
        ],
        compiler_params=pltpu.CompilerParams(use_tc_tiling_on_sc=False),
    )
    return f(x, h01p, t0f, t1f)


def kernel(x, table0, table1, h0, h1):
    c4 = jnp.arange(N_CHUNKS, dtype=jnp.int32)
    h0p = jnp.pad(h0, ((0, VPAD - VOCAB), (0, 0)))
    h1p = jnp.pad(h1, ((0, VPAD - VOCAB), (0, 0)))
    lo = h0p * N_CHUNKS + c4[None, :]
    hi = h1p * N_CHUNKS + c4[None, :]
    h01p = ((lo | (hi << 16)).T.reshape(N_CHUNKS, VPAD // 128, 128)
            .transpose(1, 0, 2).reshape(VPAD * N_CHUNKS))
    t0f = table0.reshape(ROWS * N_CHUNKS, CHUNK_SIZE)
    t1f = table1.reshape(ROWS * N_CHUNKS, CHUNK_SIZE)
    return _run(x, h01p, t0f, t1f)


# speedup vs baseline: 1.2654x; 1.2654x over previous
import jax
import jax.numpy as jnp
from jax import lax
from jax.experimental import pallas as pl
from jax.experimental.pallas import tpu as pltpu
from jax.experimental.pallas import tpu_sc as plsc

VOCAB = 1000000
ROWS = 4096
CHUNK_SIZE = 16
N_CHUNKS = 4
BATCH = 16384

NC = 2
NS = 16
NW = NC * NS
BPW = BATCH // NW
PW = BPW * N_CHUNKS
NSLICE = PW // 128


def _body(x_hbm, h01_hbm, t0_hbm, t1_hbm, out_hbm,
          xv, cw, ti0, ti1, g, sem):
    wid = lax.axis_index("s") * NC + lax.axis_index("c")
    base = wid * BPW

    pltpu.sync_copy(x_hbm.at[pl.ds(base, BPW)], xv)

    copies = []
    for j in range(NSLICE):
        c, jj = divmod(j, NSLICE // N_CHUNKS)
        idx = xv.at[pl.ds(jj * 128, 128)]
        copies.append(pltpu.async_copy(
            h01_hbm.at[c].at[idx], cw.at[pl.ds(j * 128, 128)], sem))
    for cp in copies:
        cp.wait()

    def unpack_body(k, _):
        sl = pl.ds(k * 16, 16)
        w = cw[sl]
        ti0[sl] = lax.bitwise_and(w, 0xFFFF)
        ti1[sl] = lax.shift_right_logical(w, 16)
        return 0
    lax.fori_loop(0, PW // 16, unpack_body, 0, unroll=2)

    copies = []
    for j in range(NSLICE):
        copies.append(pltpu.async_copy(
            t0_hbm.at[ti0.at[pl.ds(j * 128, 128)]],
            g.at[pl.ds(j * 128, 128)], sem))
    for cp in copies:
        cp.wait()

    copies = []
    for j in range(NSLICE):
        copies.append(pltpu.async_copy(
            t1_hbm.at[ti1.at[pl.ds(j * 128, 128)]],
            g.at[pl.ds(j * 128, 128)], sem, add=True))
    for cp in copies:
        cp.wait()

    for c in range(N_CHUNKS):
        pltpu.sync_copy(g.at[pl.ds(c * BPW, BPW), :],
                        out_hbm.at[pl.ds(base, BPW),
                                   pl.ds(c * CHUNK_SIZE, CHUNK_SIZE)])


@jax.jit
def _run(x, h01p, t0f, t1f):
    mesh = plsc.VectorSubcoreMesh(core_axis_name="c", subcore_axis_name="s")
    f = pl.kernel(
        _body,
        out_type=jax.ShapeDtypeStruct((BATCH, N_CHUNKS * CHUNK_SIZE),
                                      jnp.float32),
        mesh=mesh,
        scratch_types=[
            pltpu.VMEM((BPW,), jnp.int32),
            pltpu.VMEM((PW,), jnp.int32),
            pltpu.VMEM((PW,), jnp.int32),
            pltpu.VMEM((PW,), jnp.int32),
            pltpu.VMEM((PW, CHUNK_SIZE), jnp.float32),
            pltpu.SemaphoreType.DMA,
        ],
        compiler_params=pltpu.CompilerParams(use_tc_tiling_on_sc=False),
    )
    return f(x, h01p, t0f, t1f)


def kernel(x, table0, table1, h0, h1):
    c4 = jnp.arange(N_CHUNKS, dtype=jnp.int32)
    lo = h0 + c4[None, :] * ROWS
    hi = h1 + c4[None, :] * ROWS
    h01p = (lo | (hi << 16)).T
    t0f = table0.transpose(1, 0, 2).reshape(ROWS * N_CHUNKS, CHUNK_SIZE)
    t1f = table1.transpose(1, 0, 2).reshape(ROWS * N_CHUNKS, CHUNK_SIZE)
    return _run(x, h01p, t0f, t1f)


# speedup vs baseline: 1.2760x; 1.0084x over previous
import jax
import jax.numpy as jnp
from jax import lax
from jax.experimental import pallas as pl
from jax.experimental.pallas import tpu as pltpu
from jax.experimental.pallas import tpu_sc as plsc

VOCAB = 1000000
ROWS = 4096
CHUNK_SIZE = 16
N_CHUNKS = 4
BATCH = 16384

NC = 2
NS = 16
NW = NC * NS
BPW = BATCH // NW
PW = BPW * N_CHUNKS
NSLICE = PW // 128


def _body(x_hbm, h01_hbm, t0_hbm, t1_hbm, out_hbm,
          xv, cw, ti0, ti1, g, sem):
    wid = lax.axis_index("s") * NC + lax.axis_index("c")
    base = wid * BPW

    pltpu.sync_copy(x_hbm.at[pl.ds(base, BPW)], xv)

    cw_copies = []
    for j in range(NSLICE):
        c, jj = divmod(j, NSLICE // N_CHUNKS)
        idx = xv.at[pl.ds(jj * 128, 128)]
        cw_copies.append(pltpu.async_copy(
            h01_hbm.at[c].at[idx], cw.at[pl.ds(j * 128, 128)], sem))

    t0_copies = []
    for j in range(NSLICE):
        cw_copies[j].wait()
        for k in range(8):
            sl = pl.ds((j * 8 + k) * 16, 16)
            w = cw[sl]
            ti0[sl] = lax.bitwise_and(w, 0xFFFF)
            ti1[sl] = lax.shift_right_logical(w, 16)
        t0_copies.append(pltpu.async_copy(
            t0_hbm.at[ti0.at[pl.ds(j * 128, 128)]],
            g.at[pl.ds(j * 128, 128)], sem))

    t1_copies = []
    for j in range(NSLICE):
        t0_copies[j].wait()
        t1_copies.append(pltpu.async_copy(
            t1_hbm.at[ti1.at[pl.ds(j * 128, 128)]],
            g.at[pl.ds(j * 128, 128)], sem, add=True))
    for cp in t1_copies:
        cp.wait()

    for c in range(N_CHUNKS):
        pltpu.sync_copy(g.at[pl.ds(c * BPW, BPW), :],
                        out_hbm.at[pl.ds(base, BPW),
                                   pl.ds(c * CHUNK_SIZE, CHUNK_SIZE)])


@jax.jit
def _run(x, h01p, t0f, t1f):
    mesh = plsc.VectorSubcoreMesh(core_axis_name="c", subcore_axis_name="s")
    f = pl.kernel(
        _body,
        out_type=jax.ShapeDtypeStruct((BATCH, N_CHUNKS * CHUNK_SIZE),
                                      jnp.float32),
        mesh=mesh,
        scratch_types=[
            pltpu.VMEM((BPW,), jnp.int32),
            pltpu.VMEM((PW,), jnp.int32),
            pltpu.VMEM((PW,), jnp.int32),
            pltpu.VMEM((PW,), jnp.int32),
            pltpu.VMEM((PW, CHUNK_SIZE), jnp.float32),
            pltpu.SemaphoreType.DMA,
        ],
        compiler_params=pltpu.CompilerParams(use_tc_tiling_on_sc=False),
    )
    return f(x, h01p, t0f, t1f)


def kernel(x, table0, table1, h0, h1):
    c4 = jnp.arange(N_CHUNKS, dtype=jnp.int32)
    lo = h0 + c4[None, :] * ROWS
    hi = h1 + c4[None, :] * ROWS
    h01p = (lo | (hi << 16)).T
    t0f = table0.transpose(1, 0, 2).reshape(ROWS * N_CHUNKS, CHUNK_SIZE)
    t1f = table1.transpose(1, 0, 2).reshape(ROWS * N_CHUNKS, CHUNK_SIZE)
    return _run(x, h01p, t0f, t1f)
